# baseline (speedup 1.0000x reference)
import jax
import jax.numpy as jnp
from jax.experimental import pallas as pl
from jax.experimental.pallas import tpu as pltpu

N = 10000
D = 256
NEG_SLOPE = 0.2


def _final_mm_body(fit_ref, w_ref, b_ref, o_ref):
    o_ref[...] = jnp.dot(fit_ref[...], w_ref[...],
                         preferred_element_type=jnp.float32) + b_ref[...]


def _final_mm(fit, W, b):
    n, c = fit.shape
    blk = 2000
    return pl.pallas_call(
        _final_mm_body,
        grid=(n // blk,),
        in_specs=[
            pl.BlockSpec((blk, c), lambda i: (i, 0)),
            pl.BlockSpec((c, c), lambda i: (0, 0)),
            pl.BlockSpec((1, c), lambda i: (0, 0)),
        ],
        out_specs=pl.BlockSpec((blk, c), lambda i: (i, 0)),
        out_shape=jax.ShapeDtypeStruct((n, c), jnp.float32),
    )(fit, W, b.reshape(1, c))


def _seq_edges(n):
    r = jnp.arange(n - 1, dtype=jnp.int32)
    e1 = jnp.stack([r, r + 1], axis=0)
    e2 = jnp.stack([r + 1, r], axis=0)
    return jnp.concatenate([e1, e2], axis=1)


def _seg_softmax(score, index, n):
    mx = jax.ops.segment_max(score, index, num_segments=n)
    mx = jnp.where(jnp.isfinite(mx), mx, 0.0)
    e = jnp.exp(score - mx[index])
    s = jax.ops.segment_sum(e, index, num_segments=n)
    return e / (s[index] + 1e-16)


def _attention(x, edge_index, lin_W, lin_b, att_W, att_b, n):
    src, dst = edge_index[0], edge_index[1]
    x_pool_j = x[src]
    x_q = jax.ops.segment_max(x_pool_j, dst, num_segments=n)
    x_q = jnp.where(jnp.isfinite(x_q), x_q, 0.0)
    x_q = (x_q @ lin_W + lin_b)[dst]
    score = (jnp.concatenate([x_q, x_pool_j], axis=-1) @ att_W + att_b).reshape(-1)
    score = jnp.where(score >= 0, score, NEG_SLOPE * score)
    return _seg_softmax(score, dst, n)


def _leconv(x, edge_index, W1, b1, W2, W3, b3, n):
    a = x @ W1 + b1
    b = x @ W2
    src, dst = edge_index[0], edge_index[1]
    msg = a[src] - b[dst]
    out = jax.ops.segment_sum(msg, dst, num_segments=n)
    return out + (x @ W3 + b3)


def kernel(x, edge_index,
           lin_top_W, lin_top_b, att_top_W, att_top_b,
           lin_seq_W, lin_seq_b, att_seq_W, att_seq_b,
           le_top_W1, le_top_b1, le_top_W2, le_top_W3, le_top_b3,
           le_seq_W1, le_seq_b1, le_seq_W2, le_seq_W3, le_seq_b3,
           se_fc1_W, se_fc1_b, se_fc2_W, se_fc2_b,
           final_W, final_b):
    n = x.shape[0]
    C = final_W.shape[0]
    ei_seq = _seq_edges(n)
    score_top = _attention(x, edge_index, lin_top_W, lin_top_b, att_top_W, att_top_b, n)
    score_seq = _attention(x, ei_seq, lin_seq_W, lin_seq_b, att_seq_W, att_seq_b, n)
    v_top = x[edge_index[0]] * score_top[:, None]
    x_top = jax.ops.segment_sum(v_top, edge_index[1], num_segments=n)
    v_seq = x[ei_seq[0]] * score_seq[:, None]
    x_seq = jax.ops.segment_sum(v_seq, ei_seq[1], num_segments=n)
    xc = jnp.concatenate([x_top, x_seq], axis=-1)
    fit_top = _leconv(xc, edge_index, le_top_W1, le_top_b1, le_top_W2, le_top_W3, le_top_b3, n)
    fit_seq = _leconv(xc, ei_seq, le_seq_W1, le_seq_b1, le_seq_W2, le_seq_W3, le_seq_b3, n)
    fit = jnp.concatenate([fit_top, fit_seq], axis=-1)
    squeeze = fit.reshape(1, -1, C).mean(axis=1)
    s1 = jax.nn.relu(squeeze @ se_fc1_W + se_fc1_b)
    s2 = jax.nn.sigmoid(s1 @ se_fc2_W + se_fc2_b)
    fit = (fit * s2).reshape(-1, C)
    fit = jax.nn.relu(fit)
    return _final_mm(fit, final_W, final_b)


# speedup vs baseline: 2.4365x; 2.4365x over previous
import functools

import jax
import jax.numpy as jnp
from jax import lax
from jax.experimental import pallas as pl
from jax.experimental.pallas import tpu as pltpu
from jax.experimental.pallas import tpu_sc as plsc

N = 10000
D = 256
NEG_SLOPE = 0.2

NW = 32
NPT = 320
NPAD = NW * NPT
NSC = 16 * NPT
CAP = 16384
ECH = 8000
EB = 128
E_TOP = 160000


def _sc_leconv_agg(lists, cnts, a):
    F = a.shape[1]
    mesh = plsc.VectorSubcoreMesh(core_axis_name="c", subcore_axis_name="s")

    @functools.partial(
        pl.kernel,
        out_type=[
            jax.ShapeDtypeStruct((NPAD, F), jnp.float32),
        ],
        mesh=mesh,
        compiler_params=pltpu.CompilerParams(needs_layout_passes=False),
        scratch_types=[
            pltpu.VMEM((CAP,), jnp.int32),
            pltpu.VMEM((16,), jnp.int32),
            pltpu.VMEM((EB,), jnp.int32),
            pltpu.VMEM((EB,), jnp.int32),
            pltpu.VMEM((EB, 128), jnp.float32),
            pltpu.VMEM((64, 128), jnp.float32),
            pltpu.VMEM_SHARED((NSC + 8, 128), jnp.float32),
            pltpu.SemaphoreType.DMA,
        ],
    )
    def k(lists_hbm, cnts_hbm, a_hbm, agg_hbm,
          list_v, cntb, idxb, sclsb, rows, zbuf, acc_sh, sem):
        c = lax.axis_index("c")
        s = lax.axis_index("s")
        wid = c * 16 + s
        base = wid * NPT
        lanes = lax.iota(jnp.int32, 16)
        zeros16 = jnp.zeros((16,), jnp.float32)

        pltpu.sync_copy(lists_hbm.at[wid], list_v)
        pltpu.sync_copy(cnts_hbm.at[wid], cntb)
        cnt = cntb[pl.ds(0, 16)][0]

        def z1(r, _):
            for kk in range(128 // 16):
                zbuf[r, pl.ds(16 * kk, 16)] = zeros16
            return 0
        lax.fori_loop(0, 64, z1, 0)

        def z3(t, _):
            pltpu.sync_copy(zbuf, acc_sh.at[pl.ds(s * NPT + t * 64, 64)])
            return 0
        lax.fori_loop(0, NPT // 64, z3, 0)

        @pl.when(s == 0)
        def _():
            pltpu.sync_copy(zbuf.at[pl.ds(0, 8)], acc_sh.at[pl.ds(NSC, 8)])

        def batch(b, _):
            def bld(g, _):
                pk = list_v[pl.ds(b * EB + g * 16, 16)]
                valid = (b * EB + g * 16 + lanes) < cnt
                sv = pk & jnp.int32(16383)
                t = lax.shift_right_logical(pk, 14)
                idxb[pl.ds(g * 16, 16)] = jnp.where(valid, sv, 0)
                sclsb[pl.ds(g * 16, 16)] = jnp.where(valid, s * NPT + t, NSC)
                return 0
            lax.fori_loop(0, EB // 16, bld, 0)
            pltpu.async_copy(a_hbm.at[idxb], rows, sem).wait()
            pltpu.sync_copy(rows, acc_sh.at[sclsb], add=True)
            return 0
        lax.fori_loop(0, (cnt + EB - 1) // EB, batch, 0)

        def outp(t, _):
            pltpu.sync_copy(acc_sh.at[pl.ds(s * NPT + t * 64, 64)],
                            rows.at[pl.ds(0, 64)])
            pltpu.sync_copy(rows.at[pl.ds(0, 64)],
                            agg_hbm.at[pl.ds(base + t * 64, 64)])
            return 0
        lax.fori_loop(0, NPT // 64, outp, 0)

    return k(lists, cnts, a)


ECH_A = 4000
EB_A = 32


def _sc_attention_top(src, dst, x, weff, beff16, sj_pad):
    mesh = plsc.VectorSubcoreMesh(core_axis_name="c", subcore_axis_name="s")
    ninf = jnp.float32(float("-inf"))

    @functools.partial(
        pl.kernel,
        out_type=[
            jax.ShapeDtypeStruct((NPAD, 256), jnp.float32),
            jax.ShapeDtypeStruct((NPAD,), jnp.float32),
            jax.ShapeDtypeStruct((NW, CAP), jnp.int32),
            jax.ShapeDtypeStruct((NW, 16), jnp.int32),
        ],
        mesh=mesh,
        compiler_params=pltpu.CompilerParams(needs_layout_passes=False),
        scratch_types=[
            pltpu.VMEM((CAP,), jnp.int32),
            pltpu.VMEM((ECH_A,), jnp.int32),
            pltpu.VMEM((ECH_A,), jnp.int32),
            pltpu.VMEM((NPAD,), jnp.float32),
            pltpu.VMEM((NPT, 256), jnp.float32),
            pltpu.VMEM((EB_A, 256), jnp.float32),
            pltpu.VMEM((NPT,), jnp.float32),
            pltpu.VMEM((NPT,), jnp.float32),
            pltpu.VMEM((NPT,), jnp.float32),
            pltpu.VMEM((EB_A,), jnp.float32),
            pltpu.VMEM((EB_A,), jnp.int32),
            pltpu.VMEM((EB_A,), jnp.int32),
            pltpu.VMEM((256,), jnp.float32),
            pltpu.VMEM((16,), jnp.float32),
            pltpu.VMEM((16,), jnp.int32),
            pltpu.SemaphoreType.DMA,
        ],
    )
    def k(src_hbm, dst_hbm, x_hbm, weff_hbm, beff_hbm, sj_hbm,
          xtop_hbm, deg_hbm, lists_hbm, cnts_hbm,
          list_v, srcc, dstc, sj_v, acc, rows, sq_v, ssum_v, deg_v,
          alphab, idxb, tb, weff_v, beff_v, cntb, sem):
        c = lax.axis_index("c")
        s = lax.axis_index("s")
        wid = c * 16 + s
        base = wid * NPT
        lanes = lax.iota(jnp.int32, 16)
        zeros16 = jnp.zeros((16,), jnp.float32)
        ninf16 = jnp.full((16,), ninf, jnp.float32)

        pltpu.sync_copy(sj_hbm, sj_v)
        pltpu.sync_copy(weff_hbm, weff_v)
        pltpu.sync_copy(beff_hbm, beff_v)

        def z1(g, _):
            deg_v[pl.ds(g * 16, 16)] = zeros16
            ssum_v[pl.ds(g * 16, 16)] = zeros16
            sq_v[pl.ds(g * 16, 16)] = zeros16
            return 0
        lax.fori_loop(0, NPT // 16, z1, 0)

        def z2(r, _):
            for kk in range(16):
                acc[r, pl.ds(16 * kk, 16)] = ninf16
            return 0
        lax.fori_loop(0, NPT, z2, 0)

        def chunk(ci, cnt):
            pltpu.sync_copy(src_hbm.at[pl.ds(ci * ECH_A, ECH_A)], srcc)
            pltpu.sync_copy(dst_hbm.at[pl.ds(ci * ECH_A, ECH_A)], dstc)

            def inner(i, cnt):
                sv = srcc[pl.ds(i * 16, 16)]
                dv = dstc[pl.ds(i * 16, 16)]
                t = dv - base
                m = (t >= 0) & (t < NPT)
                packed = sv | lax.shift_left(t, 14)
                cnt_c = jnp.minimum(cnt, CAP - 16)
                cs = plsc.cumsum(m.astype(jnp.int32))
                plsc.store_scatter(list_v, [cnt_c + cs - 1], packed, mask=m)
                return cnt_c + jnp.sum(m.astype(jnp.int32))
            return lax.fori_loop(0, ECH_A // 16, inner, cnt)
        cnt = lax.fori_loop(0, E_TOP // ECH_A, chunk, jnp.int32(0))

        pltpu.sync_copy(list_v, lists_hbm.at[wid])
        cntb[pl.ds(0, 16)] = jnp.full((16,), cnt, jnp.int32)
        pltpu.sync_copy(cntb, cnts_hbm.at[wid])

        ones16 = jnp.ones((16,), jnp.float32)

        def dacc(g, _):
            pk = list_v[pl.ds(g * 16, 16)]
            valid = (g * 16 + lanes) < cnt
            t = lax.shift_right_logical(pk, 14)
            t = jnp.where(valid, t, NPT - 1)
            plsc.addupdate_scatter(deg_v, [t], ones16, mask=valid)
            return 0
        lax.fori_loop(0, (cnt + 15) // 16, dacc, 0)

        def batchA(b, _):
            def bld(g, _):
                pk = list_v[pl.ds(b * EB_A + g * 16, 16)]
                valid = (b * EB_A + g * 16 + lanes) < cnt
                idxb[pl.ds(g * 16, 16)] = jnp.where(valid, pk & jnp.int32(16383), 0)
                tb[pl.ds(g * 16, 16)] = lax.shift_right_logical(pk, 14)
                return 0
            lax.fori_loop(0, EB_A // 16, bld, 0)
            pltpu.async_copy(x_hbm.at[idxb], rows, sem).wait()
            nr = jnp.minimum(EB_A, cnt - b * EB_A)

            def rowacc(r, _):
                ts = plsc.load_gather(tb, [jnp.full((16,), r, jnp.int32)])[0]
                for kk in range(16):
                    sl = pl.ds(16 * kk, 16)
                    acc[ts, sl] = jnp.maximum(acc[ts, sl], rows[r, sl])
                return 0
            lax.fori_loop(0, nr, rowacc, 0)
            return 0
        lax.fori_loop(0, (cnt + EB_A - 1) // EB_A, batchA, 0)

        beff_s = beff_v[pl.ds(0, 16)][0]

        def nodeB(r, _):
            accum = zeros16
            for kk in range(16):
                sl = pl.ds(16 * kk, 16)
                row = acc[r, sl]
                rowf = jnp.where(row > ninf, row, 0.0)
                accum = accum + rowf * weff_v[sl]
            sq_s = jnp.sum(accum) + beff_s
            plsc.store_scatter(sq_v, [jnp.full((16,), r, jnp.int32)],
                               jnp.full((16,), sq_s, jnp.float32),
                               mask=lanes < 1)
            return 0
        lax.fori_loop(0, NPT, nodeB, 0)

        def grpB2(g, _):
            pk = list_v[pl.ds(g * 16, 16)]
            valid = (g * 16 + lanes) < cnt
            sv = jnp.where(valid, pk & jnp.int32(16383), 0)
            t = jnp.where(valid, lax.shift_right_logical(pk, 14), 0)
            sc = plsc.load_gather(sj_v, [sv]) + plsc.load_gather(sq_v, [t])
            sc = jnp.where(sc >= 0, sc, NEG_SLOPE * sc)
            ev = jnp.exp(sc)
            plsc.addupdate_scatter(ssum_v, [t], ev, mask=valid)
            return 0
        lax.fori_loop(0, (cnt + 15) // 16, grpB2, 0)

        def zc(r, _):
            for kk in range(16):
                acc[r, pl.ds(16 * kk, 16)] = zeros16
            return 0
        lax.fori_loop(0, NPT, zc, 0)

        def batchC(b, _):
            def bldC(g, _):
                pk = list_v[pl.ds(b * EB_A + g * 16, 16)]
                valid = (b * EB_A + g * 16 + lanes) < cnt
                sv = jnp.where(valid, pk & jnp.int32(16383), 0)
                t = jnp.where(valid, lax.shift_right_logical(pk, 14), 0)
                sc = plsc.load_gather(sj_v, [sv]) + plsc.load_gather(sq_v, [t])
                sc = jnp.where(sc >= 0, sc, NEG_SLOPE * sc)
                ev = jnp.exp(sc)
                ssv = plsc.load_gather(ssum_v, [t])
                alphab[pl.ds(g * 16, 16)] = ev / (ssv + 1e-16)
                idxb[pl.ds(g * 16, 16)] = sv
                tb[pl.ds(g * 16, 16)] = t
                return 0
            lax.fori_loop(0, EB_A // 16, bldC, 0)
            pltpu.async_copy(x_hbm.at[idxb], rows, sem).wait()
            nr = jnp.minimum(EB_A, cnt - b * EB_A)

            def rowadd(r, _):
                av = plsc.load_gather(alphab, [jnp.full((16,), r, jnp.int32)])
                ts = plsc.load_gather(tb, [jnp.full((16,), r, jnp.int32)])[0]
                for kk in range(16):
                    sl = pl.ds(16 * kk, 16)
                    acc[ts, sl] = acc[ts, sl] + rows[r, sl] * av
                return 0
            lax.fori_loop(0, nr, rowadd, 0)
            return 0
        lax.fori_loop(0, (cnt + EB_A - 1) // EB_A, batchC, 0)

        def outX(t, _):
            pltpu.sync_copy(acc.at[pl.ds(t * EB_A, EB_A)],
                            xtop_hbm.at[pl.ds(base + t * EB_A, EB_A)])
            return 0
        lax.fori_loop(0, NPT // EB_A, outX, 0)
        pltpu.sync_copy(deg_v, deg_hbm.at[pl.ds(base, NPT)])

    return k(src, dst, x, weff, beff16, sj_pad)


def _final_mm_body(fit_ref, w_ref, b_ref, o_ref):
    o_ref[...] = jnp.dot(fit_ref[...], w_ref[...],
                         preferred_element_type=jnp.float32) + b_ref[...]


def _final_mm(fit, W, b):
    n, c = fit.shape
    blk = 2000
    return pl.pallas_call(
        _final_mm_body,
        grid=(n // blk,),
        in_specs=[
            pl.BlockSpec((blk, c), lambda i: (i, 0)),
            pl.BlockSpec((c, c), lambda i: (0, 0)),
            pl.BlockSpec((1, c), lambda i: (0, 0)),
        ],
        out_specs=pl.BlockSpec((blk, c), lambda i: (i, 0)),
        out_shape=jax.ShapeDtypeStruct((n, c), jnp.float32),
    )(fit, W, b.reshape(1, c))


def _seq_edges(n):
    r = jnp.arange(n - 1, dtype=jnp.int32)
    e1 = jnp.stack([r, r + 1], axis=0)
    e2 = jnp.stack([r + 1, r], axis=0)
    return jnp.concatenate([e1, e2], axis=1)


def _seg_softmax(score, index, n):
    mx = jax.ops.segment_max(score, index, num_segments=n)
    mx = jnp.where(jnp.isfinite(mx), mx, 0.0)
    e = jnp.exp(score - mx[index])
    s = jax.ops.segment_sum(e, index, num_segments=n)
    return e / (s[index] + 1e-16)


def _attention(x, edge_index, lin_W, lin_b, att_W, att_b, n):
    src, dst = edge_index[0], edge_index[1]
    x_pool_j = x[src]
    x_q = jax.ops.segment_max(x_pool_j, dst, num_segments=n)
    x_q = jnp.where(jnp.isfinite(x_q), x_q, 0.0)
    x_q = (x_q @ lin_W + lin_b)[dst]
    score = (jnp.concatenate([x_q, x_pool_j], axis=-1) @ att_W + att_b).reshape(-1)
    score = jnp.where(score >= 0, score, NEG_SLOPE * score)
    return _seg_softmax(score, dst, n)


def _leconv(x, edge_index, W1, b1, W2, W3, b3, n):
    a = x @ W1 + b1
    b = x @ W2
    src, dst = edge_index[0], edge_index[1]
    msg = a[src] - b[dst]
    out = jax.ops.segment_sum(msg, dst, num_segments=n)
    return out + (x @ W3 + b3)


def kernel(x, edge_index,
           lin_top_W, lin_top_b, att_top_W, att_top_b,
           lin_seq_W, lin_seq_b, att_seq_W, att_seq_b,
           le_top_W1, le_top_b1, le_top_W2, le_top_W3, le_top_b3,
           le_seq_W1, le_seq_b1, le_seq_W2, le_seq_W3, le_seq_b3,
           se_fc1_W, se_fc1_b, se_fc2_W, se_fc2_b,
           final_W, final_b):
    n = x.shape[0]
    C = final_W.shape[0]
    ei_seq = _seq_edges(n)
    wq = att_top_W[:D, 0]
    weff = lin_top_W @ wq
    beff16 = jnp.full((16,), lin_top_b @ wq + att_top_b[0], jnp.float32)
    sj = x @ att_top_W[D:, 0]
    sj_pad = jnp.concatenate([sj, jnp.zeros((NPAD - n,), jnp.float32)])
    xtop_pad, deg_pad, lists, cnts = _sc_attention_top(
        edge_index[0], edge_index[1], x, weff, beff16, sj_pad)
    x_top = xtop_pad[:n]
    score_seq = _attention(x, ei_seq, lin_seq_W, lin_seq_b, att_seq_W, att_seq_b, n)
    v_seq = x[ei_seq[0]] * score_seq[:, None]
    x_seq = jax.ops.segment_sum(v_seq, ei_seq[1], num_segments=n)
    xc = jnp.concatenate([x_top, x_seq], axis=-1)
    a_top = xc @ le_top_W1 + le_top_b1
    agg_pad = _sc_leconv_agg(lists, cnts, a_top)[0]
    fit_top = (agg_pad[:n] - deg_pad[:n, None] * (xc @ le_top_W2)
               + (xc @ le_top_W3 + le_top_b3))
    fit_seq = _leconv(xc, ei_seq, le_seq_W1, le_seq_b1, le_seq_W2, le_seq_W3, le_seq_b3, n)
    fit = jnp.concatenate([fit_top, fit_seq], axis=-1)
    squeeze = fit.reshape(1, -1, C).mean(axis=1)
    s1 = jax.nn.relu(squeeze @ se_fc1_W + se_fc1_b)
    s2 = jax.nn.sigmoid(s1 @ se_fc2_W + se_fc2_b)
    fit = (fit * s2).reshape(-1, C)
    fit = jax.nn.relu(fit)
    return _final_mm(fit, final_W, final_b)


# speedup vs baseline: 3.2266x; 1.3243x over previous
import functools

import jax
import jax.numpy as jnp
from jax import lax
from jax.experimental import pallas as pl
from jax.experimental.pallas import tpu as pltpu
from jax.experimental.pallas import tpu_sc as plsc

N = 10000
D = 256
NEG_SLOPE = 0.2

NW = 32
NPT = 320
NPAD = NW * NPT
NSC = 16 * NPT
CAP = 16384
ECH = 8000
EB = 128
E_TOP = 160000


def _sc_leconv_agg(lists, cnts, a):
    F = a.shape[1]
    mesh = plsc.VectorSubcoreMesh(core_axis_name="c", subcore_axis_name="s")

    @functools.partial(
        pl.kernel,
        out_type=[
            jax.ShapeDtypeStruct((NPAD, F), jnp.float32),
        ],
        mesh=mesh,
        compiler_params=pltpu.CompilerParams(needs_layout_passes=False),
        scratch_types=[
            pltpu.VMEM((CAP,), jnp.int32),
            pltpu.VMEM((16,), jnp.int32),
            pltpu.VMEM((EB,), jnp.int32),
            pltpu.VMEM((EB,), jnp.int32),
            pltpu.VMEM((EB, 128), jnp.float32),
            pltpu.VMEM((64, 128), jnp.float32),
            pltpu.VMEM_SHARED((NSC + 8, 128), jnp.float32),
            pltpu.SemaphoreType.DMA,
        ],
    )
    def k(lists_hbm, cnts_hbm, a_hbm, agg_hbm,
          list_v, cntb, idxb, sclsb, rows, zbuf, acc_sh, sem):
        c = lax.axis_index("c")
        s = lax.axis_index("s")
        wid = c * 16 + s
        base = wid * NPT
        lanes = lax.iota(jnp.int32, 16)
        zeros16 = jnp.zeros((16,), jnp.float32)

        pltpu.sync_copy(lists_hbm.at[wid], list_v)
        pltpu.sync_copy(cnts_hbm.at[wid], cntb)
        cnt = cntb[pl.ds(0, 16)][0]

        def z1(r, _):
            for kk in range(128 // 16):
                zbuf[r, pl.ds(16 * kk, 16)] = zeros16
            return 0
        lax.fori_loop(0, 64, z1, 0)

        def z3(t, _):
            pltpu.sync_copy(zbuf, acc_sh.at[pl.ds(s * NPT + t * 64, 64)])
            return 0
        lax.fori_loop(0, NPT // 64, z3, 0)

        @pl.when(s == 0)
        def _():
            pltpu.sync_copy(zbuf.at[pl.ds(0, 8)], acc_sh.at[pl.ds(NSC, 8)])

        def batch(b, _):
            def bld(g, _):
                pk = list_v[pl.ds(b * EB + g * 16, 16)]
                valid = (b * EB + g * 16 + lanes) < cnt
                sv = pk & jnp.int32(16383)
                t = lax.shift_right_logical(pk, 14)
                idxb[pl.ds(g * 16, 16)] = jnp.where(valid, sv, 0)
                sclsb[pl.ds(g * 16, 16)] = jnp.where(valid, s * NPT + t, NSC)
                return 0
            lax.fori_loop(0, EB // 16, bld, 0)
            pltpu.async_copy(a_hbm.at[idxb], rows, sem).wait()
            pltpu.sync_copy(rows, acc_sh.at[sclsb], add=True)
            return 0
        lax.fori_loop(0, (cnt + EB - 1) // EB, batch, 0)

        def outp(t, _):
            pltpu.sync_copy(acc_sh.at[pl.ds(s * NPT + t * 64, 64)],
                            rows.at[pl.ds(0, 64)])
            pltpu.sync_copy(rows.at[pl.ds(0, 64)],
                            agg_hbm.at[pl.ds(base + t * 64, 64)])
            return 0
        lax.fori_loop(0, NPT // 64, outp, 0)

    return k(lists, cnts, a)


ECH_A = 4000
EB_A = 32


def _sc_attention_top(src, dst, x, weff, beff16, sj_pad):
    mesh = plsc.VectorSubcoreMesh(core_axis_name="c", subcore_axis_name="s")
    ninf = jnp.float32(float("-inf"))

    @functools.partial(
        pl.kernel,
        out_type=[
            jax.ShapeDtypeStruct((NPAD, 256), jnp.float32),
            jax.ShapeDtypeStruct((NPAD,), jnp.float32),
            jax.ShapeDtypeStruct((NW, CAP), jnp.int32),
            jax.ShapeDtypeStruct((NW, 16), jnp.int32),
        ],
        mesh=mesh,
        compiler_params=pltpu.CompilerParams(needs_layout_passes=False),
        scratch_types=[
            pltpu.VMEM((CAP,), jnp.int32),
            pltpu.VMEM((ECH_A,), jnp.int32),
            pltpu.VMEM((ECH_A,), jnp.int32),
            pltpu.VMEM((NPAD,), jnp.float32),
            pltpu.VMEM((NPT, 256), jnp.float32),
            pltpu.VMEM((EB_A, 256), jnp.float32),
            pltpu.VMEM((NPT,), jnp.float32),
            pltpu.VMEM((NPT,), jnp.float32),
            pltpu.VMEM((NPT,), jnp.float32),
            pltpu.VMEM((EB_A,), jnp.float32),
            pltpu.VMEM((EB_A,), jnp.int32),
            pltpu.VMEM((EB_A,), jnp.int32),
            pltpu.VMEM((256,), jnp.float32),
            pltpu.VMEM((16,), jnp.float32),
            pltpu.VMEM((16,), jnp.int32),
            pltpu.SemaphoreType.DMA,
        ],
    )
    def k(src_hbm, dst_hbm, x_hbm, weff_hbm, beff_hbm, sj_hbm,
          xtop_hbm, deg_hbm, lists_hbm, cnts_hbm,
          list_v, srcc, dstc, sj_v, acc, rows, sq_v, ssum_v, deg_v,
          alphab, idxb, tb, weff_v, beff_v, cntb, sem):
        c = lax.axis_index("c")
        s = lax.axis_index("s")
        wid = c * 16 + s
        base = wid * NPT
        lanes = lax.iota(jnp.int32, 16)
        zeros16 = jnp.zeros((16,), jnp.float32)
        ninf16 = jnp.full((16,), ninf, jnp.float32)

        pltpu.sync_copy(sj_hbm, sj_v)
        pltpu.sync_copy(weff_hbm, weff_v)
        pltpu.sync_copy(beff_hbm, beff_v)

        def z1(g, _):
            deg_v[pl.ds(g * 16, 16)] = zeros16
            ssum_v[pl.ds(g * 16, 16)] = zeros16
            sq_v[pl.ds(g * 16, 16)] = zeros16
            return 0
        lax.fori_loop(0, NPT // 16, z1, 0)

        def z2(r, _):
            for kk in range(16):
                acc[r, pl.ds(16 * kk, 16)] = ninf16
            return 0
        lax.fori_loop(0, NPT, z2, 0)

        def chunk(ci, cnt):
            pltpu.sync_copy(src_hbm.at[pl.ds(ci * ECH_A, ECH_A)], srcc)
            pltpu.sync_copy(dst_hbm.at[pl.ds(ci * ECH_A, ECH_A)], dstc)

            def inner(i, cnt):
                sv = srcc[pl.ds(i * 16, 16)]
                dv = dstc[pl.ds(i * 16, 16)]
                t = dv - base
                m = (t >= 0) & (t < NPT)
                packed = sv | lax.shift_left(t, 14)
                cnt_c = jnp.minimum(cnt, CAP - 16)
                cs = plsc.cumsum(m.astype(jnp.int32))
                plsc.store_scatter(list_v, [cnt_c + cs - 1], packed, mask=m)
                return cnt_c + jnp.sum(m.astype(jnp.int32))
            return lax.fori_loop(0, ECH_A // 16, inner, cnt)
        cnt = lax.fori_loop(0, E_TOP // ECH_A, chunk, jnp.int32(0))

        pltpu.sync_copy(list_v, lists_hbm.at[wid])
        cntb[pl.ds(0, 16)] = jnp.full((16,), cnt, jnp.int32)
        pltpu.sync_copy(cntb, cnts_hbm.at[wid])

        ones16 = jnp.ones((16,), jnp.float32)

        def dacc(g, _):
            pk = list_v[pl.ds(g * 16, 16)]
            valid = (g * 16 + lanes) < cnt
            t = lax.shift_right_logical(pk, 14)
            t = jnp.where(valid, t, NPT - 1)
            plsc.addupdate_scatter(deg_v, [t], ones16, mask=valid)
            return 0
        lax.fori_loop(0, (cnt + 15) // 16, dacc, 0)

        def batchA(b, _):
            def bld(g, _):
                pk = list_v[pl.ds(b * EB_A + g * 16, 16)]
                valid = (b * EB_A + g * 16 + lanes) < cnt
                idxb[pl.ds(g * 16, 16)] = jnp.where(valid, pk & jnp.int32(16383), 0)
                tb[pl.ds(g * 16, 16)] = lax.shift_right_logical(pk, 14)
                return 0
            lax.fori_loop(0, EB_A // 16, bld, 0)
            pltpu.async_copy(x_hbm.at[idxb], rows, sem).wait()
            nr = jnp.minimum(EB_A, cnt - b * EB_A)

            def rowacc(r, _):
                ts = plsc.load_gather(tb, [jnp.full((16,), r, jnp.int32)])[0]
                for kk in range(16):
                    sl = pl.ds(16 * kk, 16)
                    acc[ts, sl] = jnp.maximum(acc[ts, sl], rows[r, sl])
                return 0
            lax.fori_loop(0, nr, rowacc, 0)
            return 0
        lax.fori_loop(0, (cnt + EB_A - 1) // EB_A, batchA, 0)

        beff_s = beff_v[pl.ds(0, 16)][0]

        def nodeB(r, _):
            accum = zeros16
            for kk in range(16):
                sl = pl.ds(16 * kk, 16)
                row = acc[r, sl]
                rowf = jnp.where(row > ninf, row, 0.0)
                accum = accum + rowf * weff_v[sl]
            sq_s = jnp.sum(accum) + beff_s
            plsc.store_scatter(sq_v, [jnp.full((16,), r, jnp.int32)],
                               jnp.full((16,), sq_s, jnp.float32),
                               mask=lanes < 1)
            return 0
        lax.fori_loop(0, NPT, nodeB, 0)

        def grpB2(g, _):
            pk = list_v[pl.ds(g * 16, 16)]
            valid = (g * 16 + lanes) < cnt
            sv = jnp.where(valid, pk & jnp.int32(16383), 0)
            t = jnp.where(valid, lax.shift_right_logical(pk, 14), 0)
            sc = plsc.load_gather(sj_v, [sv]) + plsc.load_gather(sq_v, [t])
            sc = jnp.where(sc >= 0, sc, NEG_SLOPE * sc)
            ev = jnp.exp(sc)
            plsc.addupdate_scatter(ssum_v, [t], ev, mask=valid)
            return 0
        lax.fori_loop(0, (cnt + 15) // 16, grpB2, 0)

        def zc(r, _):
            for kk in range(16):
                acc[r, pl.ds(16 * kk, 16)] = zeros16
            return 0
        lax.fori_loop(0, NPT, zc, 0)

        def batchC(b, _):
            def bldC(g, _):
                pk = list_v[pl.ds(b * EB_A + g * 16, 16)]
                valid = (b * EB_A + g * 16 + lanes) < cnt
                sv = jnp.where(valid, pk & jnp.int32(16383), 0)
                t = jnp.where(valid, lax.shift_right_logical(pk, 14), 0)
                sc = plsc.load_gather(sj_v, [sv]) + plsc.load_gather(sq_v, [t])
                sc = jnp.where(sc >= 0, sc, NEG_SLOPE * sc)
                ev = jnp.exp(sc)
                ssv = plsc.load_gather(ssum_v, [t])
                alphab[pl.ds(g * 16, 16)] = ev / (ssv + 1e-16)
                idxb[pl.ds(g * 16, 16)] = sv
                tb[pl.ds(g * 16, 16)] = t
                return 0
            lax.fori_loop(0, EB_A // 16, bldC, 0)
            pltpu.async_copy(x_hbm.at[idxb], rows, sem).wait()
            nr = jnp.minimum(EB_A, cnt - b * EB_A)

            def rowadd(r, _):
                av = plsc.load_gather(alphab, [jnp.full((16,), r, jnp.int32)])
                ts = plsc.load_gather(tb, [jnp.full((16,), r, jnp.int32)])[0]
                for kk in range(16):
                    sl = pl.ds(16 * kk, 16)
                    acc[ts, sl] = acc[ts, sl] + rows[r, sl] * av
                return 0
            lax.fori_loop(0, nr, rowadd, 0)
            return 0
        lax.fori_loop(0, (cnt + EB_A - 1) // EB_A, batchC, 0)

        def outX(t, _):
            pltpu.sync_copy(acc.at[pl.ds(t * EB_A, EB_A)],
                            xtop_hbm.at[pl.ds(base + t * EB_A, EB_A)])
            return 0
        lax.fori_loop(0, NPT // EB_A, outX, 0)
        pltpu.sync_copy(deg_v, deg_hbm.at[pl.ds(base, NPT)])

    return k(src, dst, x, weff, beff16, sj_pad)


TB = 1024
NB = NPAD // TB
NINF = float("-inf")


def _tc_seq_body(xm_ref, x_ref, xp_ref, weffs_ref, wjs_ref, wjt_ref, c_ref,
                 xseq_ref, sjt_ref):
    i = pl.program_id(0)
    xb = x_ref[...]
    xm1 = jnp.concatenate([xm_ref[TB - 1:TB, :], xb[:TB - 1, :]], axis=0)
    xp1 = jnp.concatenate([xb[1:, :], xp_ref[0:1, :]], axis=0)
    gid = i * TB + jax.lax.broadcasted_iota(jnp.int32, (TB, 1), 0)
    v1 = gid >= 1
    v2 = gid <= N - 2
    m_seq = jnp.maximum(jnp.where(v1, xm1, NINF), jnp.where(v2, xp1, NINF))
    beff = c_ref[0, 0]
    sq = jnp.dot(m_seq, weffs_ref[...], preferred_element_type=jnp.float32) + beff
    sjm1 = jnp.dot(xm1, wjs_ref[...], preferred_element_type=jnp.float32)
    sjp1 = jnp.dot(xp1, wjs_ref[...], preferred_element_type=jnp.float32)
    t1 = sq + sjm1
    t1 = jnp.where(t1 >= 0, t1, NEG_SLOPE * t1)
    t2 = sq + sjp1
    t2 = jnp.where(t2 >= 0, t2, NEG_SLOPE * t2)
    mx = jnp.maximum(jnp.where(v1, t1, NINF), jnp.where(v2, t2, NINF))
    e1 = jnp.where(v1, jnp.exp(t1 - mx), 0.0)
    e2 = jnp.where(v2, jnp.exp(t2 - mx), 0.0)
    ssum = e1 + e2 + 1e-16
    xseq_ref[...] = xm1 * (e1 / ssum) + xp1 * (e2 / ssum)
    sjt_ref[...] = jnp.dot(xb, wjt_ref[...], preferred_element_type=jnp.float32).reshape(1, TB)


def _tc_seq(xpad, weffs, wjs, wjt, beff_seq):
    carr = jnp.full((1, 128), beff_seq, jnp.float32)
    return pl.pallas_call(
        _tc_seq_body,
        grid=(NB,),
        in_specs=[
            pl.BlockSpec((TB, 256), lambda i: (jnp.maximum(i - 1, 0), 0)),
            pl.BlockSpec((TB, 256), lambda i: (i, 0)),
            pl.BlockSpec((TB, 256), lambda i: (jnp.minimum(i + 1, NB - 1), 0)),
            pl.BlockSpec((256, 1), lambda i: (0, 0)),
            pl.BlockSpec((256, 1), lambda i: (0, 0)),
            pl.BlockSpec((256, 1), lambda i: (0, 0)),
            pl.BlockSpec((1, 128), lambda i: (0, 0)),
        ],
        out_specs=[
            pl.BlockSpec((TB, 256), lambda i: (i, 0)),
            pl.BlockSpec((1, TB), lambda i: (0, i)),
        ],
        out_shape=[
            jax.ShapeDtypeStruct((NPAD, 256), jnp.float32),
            jax.ShapeDtypeStruct((1, NPAD), jnp.float32),
        ],
    )(xpad, xpad, xpad, weffs, wjs, wjt, carr)


def _tc_abc_body(xt_ref, xs_ref, wt_ref, ws_ref, b_ref, o_ref):
    o_ref[...] = (jnp.dot(xt_ref[...], wt_ref[...], preferred_element_type=jnp.float32)
                  + jnp.dot(xs_ref[...], ws_ref[...], preferred_element_type=jnp.float32)
                  + b_ref[...])


def _tc_abc(xtop, xseq, w_top_half, w_seq_half, b_all):
    return pl.pallas_call(
        _tc_abc_body,
        grid=(NB,),
        in_specs=[
            pl.BlockSpec((TB, 256), lambda i: (i, 0)),
            pl.BlockSpec((TB, 256), lambda i: (i, 0)),
            pl.BlockSpec((256, 768), lambda i: (0, 0)),
            pl.BlockSpec((256, 768), lambda i: (0, 0)),
            pl.BlockSpec((1, 768), lambda i: (0, 0)),
        ],
        out_specs=pl.BlockSpec((TB, 768), lambda i: (i, 0)),
        out_shape=jax.ShapeDtypeStruct((NPAD, 768), jnp.float32),
    )(xtop, xseq, w_top_half, w_seq_half, b_all)


def _tc_fit_body(am_ref, a_ref, ap_ref, agg_ref, degb_ref, fit_ref, msum_ref):
    i = pl.program_id(0)
    abc = a_ref[...]
    gid = i * TB + jax.lax.broadcasted_iota(jnp.int32, (TB, 1), 0)
    v1 = (gid >= 1).astype(jnp.float32)
    v2 = (gid <= N - 2).astype(jnp.float32)
    fit_t = agg_ref[...] - degb_ref[...] * abc[:, 128:256] + abc[:, 256:384]
    a_s = abc[:, 384:512]
    asm1 = jnp.concatenate([am_ref[TB - 1:TB, 384:512], a_s[:TB - 1, :]], axis=0)
    asp1 = jnp.concatenate([a_s[1:, :], ap_ref[0:1, 384:512]], axis=0)
    sum_s = v1 * asm1 + v2 * asp1
    fit_s = sum_s - (v1 + v2) * abc[:, 512:640] + abc[:, 640:768]
    fit = jnp.concatenate([fit_t, fit_s], axis=1)
    fit_ref[...] = fit
    valid = gid < N

    @pl.when(i == 0)
    def _():
        msum_ref[...] = jnp.zeros_like(msum_ref)
    msum_ref[...] += jnp.sum(jnp.where(valid, fit, 0.0), axis=0, keepdims=True)


def _tc_fit(abc, agg, degb):
    return pl.pallas_call(
        _tc_fit_body,
        grid=(NB,),
        in_specs=[
            pl.BlockSpec((TB, 768), lambda i: (jnp.maximum(i - 1, 0), 0)),
            pl.BlockSpec((TB, 768), lambda i: (i, 0)),
            pl.BlockSpec((TB, 768), lambda i: (jnp.minimum(i + 1, NB - 1), 0)),
            pl.BlockSpec((TB, 128), lambda i: (i, 0)),
            pl.BlockSpec((TB, 128), lambda i: (i, 0)),
        ],
        out_specs=[
            pl.BlockSpec((TB, 256), lambda i: (i, 0)),
            pl.BlockSpec((1, 256), lambda i: (0, 0)),
        ],
        out_shape=[
            jax.ShapeDtypeStruct((NPAD, 256), jnp.float32),
            jax.ShapeDtypeStruct((1, 256), jnp.float32),
        ],
    )(abc, abc, abc, agg, degb)


def _tc_out_body(fit_ref, msum_ref, f1_ref, b1_ref, f2_ref, b2_ref,
                 wf_ref, bf_ref, o_ref):
    sqz = msum_ref[...] * jnp.float32(1.0 / N)
    s1 = jnp.maximum(jnp.dot(sqz, f1_ref[...], preferred_element_type=jnp.float32)
                     + b1_ref[...], 0.0)
    z = jnp.dot(s1, f2_ref[...], preferred_element_type=jnp.float32) + b2_ref[...]
    s2 = 1.0 / (1.0 + jnp.exp(-z))
    fit = jnp.maximum(fit_ref[...] * s2, 0.0)
    o_ref[...] = jnp.dot(fit, wf_ref[...], preferred_element_type=jnp.float32) + bf_ref[...]


def _tc_out(fit, msum, f1, b1, f2, b2, wf, bf):
    return pl.pallas_call(
        _tc_out_body,
        grid=(NB,),
        in_specs=[
            pl.BlockSpec((TB, 256), lambda i: (i, 0)),
            pl.BlockSpec((1, 256), lambda i: (0, 0)),
            pl.BlockSpec((256, 64), lambda i: (0, 0)),
            pl.BlockSpec((1, 64), lambda i: (0, 0)),
            pl.BlockSpec((64, 256), lambda i: (0, 0)),
            pl.BlockSpec((1, 256), lambda i: (0, 0)),
            pl.BlockSpec((256, 256), lambda i: (0, 0)),
            pl.BlockSpec((1, 256), lambda i: (0, 0)),
        ],
        out_specs=pl.BlockSpec((TB, 256), lambda i: (i, 0)),
        out_shape=jax.ShapeDtypeStruct((NPAD, 256), jnp.float32),
    )(fit, msum, f1, b1, f2, b2, wf, bf)


def kernel(x, edge_index,
           lin_top_W, lin_top_b, att_top_W, att_top_b,
           lin_seq_W, lin_seq_b, att_seq_W, att_seq_b,
           le_top_W1, le_top_b1, le_top_W2, le_top_W3, le_top_b3,
           le_seq_W1, le_seq_b1, le_seq_W2, le_seq_W3, le_seq_b3,
           se_fc1_W, se_fc1_b, se_fc2_W, se_fc2_b,
           final_W, final_b):
    n = x.shape[0]
    xpad = jnp.concatenate([x, jnp.zeros((NPAD - n, D), jnp.float32)], axis=0)

    wq_t = att_top_W[:D, 0]
    weff_t = lin_top_W @ wq_t
    beff16 = jnp.full((16,), lin_top_b @ wq_t + att_top_b[0], jnp.float32)
    wq_s = att_seq_W[:D, 0]
    weffs = (lin_seq_W @ wq_s).reshape(D, 1)
    beff_s = lin_seq_b @ wq_s + att_seq_b[0]
    wjs = att_seq_W[D:, :]
    wjt = att_top_W[D:, :]

    xseq_pad, sjt = _tc_seq(xpad, weffs, wjs, wjt, beff_s)
    xtop_pad, deg_pad, lists, cnts = _sc_attention_top(
        edge_index[0], edge_index[1], x, weff_t, beff16, sjt.reshape(NPAD))

    w_top_half = jnp.concatenate(
        [le_top_W1[:D], le_top_W2[:D], le_top_W3[:D],
         le_seq_W1[:D], le_seq_W2[:D], le_seq_W3[:D]], axis=1)
    w_seq_half = jnp.concatenate(
        [le_top_W1[D:], le_top_W2[D:], le_top_W3[D:],
         le_seq_W1[D:], le_seq_W2[D:], le_seq_W3[D:]], axis=1)
    z128 = jnp.zeros((128,), jnp.float32)
    b_all = jnp.concatenate(
        [le_top_b1, z128, le_top_b3, le_seq_b1, z128, le_seq_b3]).reshape(1, 768)
    abc = _tc_abc(xtop_pad, xseq_pad, w_top_half, w_seq_half, b_all)

    agg_pad = _sc_leconv_agg(lists, cnts, abc[:, :128])[0]

    degb = jnp.broadcast_to(deg_pad[:, None], (NPAD, 128))
    fit_pad, msum = _tc_fit(abc, agg_pad, degb)

    out_pad = _tc_out(fit_pad, msum, se_fc1_W, se_fc1_b.reshape(1, -1),
                      se_fc2_W, se_fc2_b.reshape(1, -1),
                      final_W, final_b.reshape(1, -1))
    return out_pad[:n]


# speedup vs baseline: 3.8681x; 1.1988x over previous
import functools

import jax
import jax.numpy as jnp
from jax import lax
from jax.experimental import pallas as pl
from jax.experimental.pallas import tpu as pltpu
from jax.experimental.pallas import tpu_sc as plsc

N = 10000
D = 256
NEG_SLOPE = 0.2

NW = 32
NPT = 320
NPAD = NW * NPT
NSC = 16 * NPT
CAP = 10240
ECH = 8000
EB = 128
E_TOP = 160000


def _sc_leconv_agg(lists, cnts, a):
    F = a.shape[1]
    mesh = plsc.VectorSubcoreMesh(core_axis_name="c", subcore_axis_name="s")

    @functools.partial(
        pl.kernel,
        out_type=[
            jax.ShapeDtypeStruct((NPAD, F), jnp.float32),
        ],
        mesh=mesh,
        compiler_params=pltpu.CompilerParams(needs_layout_passes=False),
        scratch_types=[
            pltpu.VMEM((CAP,), jnp.int32),
            pltpu.VMEM((16,), jnp.int32),
            pltpu.VMEM((EB,), jnp.int32),
            pltpu.VMEM((EB,), jnp.int32),
            pltpu.VMEM((EB, 128), jnp.float32),
            pltpu.VMEM((64, 128), jnp.float32),
            pltpu.VMEM_SHARED((NSC + 8, 128), jnp.float32),
            pltpu.SemaphoreType.DMA,
        ],
    )
    def k(lists_hbm, cnts_hbm, a_hbm, agg_hbm,
          list_v, cntb, idxb, sclsb, rows, zbuf, acc_sh, sem):
        c = lax.axis_index("c")
        s = lax.axis_index("s")
        wid = c * 16 + s
        base = wid * NPT
        lanes = lax.iota(jnp.int32, 16)
        zeros16 = jnp.zeros((16,), jnp.float32)

        pltpu.sync_copy(lists_hbm.at[wid], list_v)
        pltpu.sync_copy(cnts_hbm.at[wid], cntb)
        cnt = cntb[pl.ds(0, 16)][0]

        def z1(r, _):
            for kk in range(128 // 16):
                zbuf[r, pl.ds(16 * kk, 16)] = zeros16
            return 0
        lax.fori_loop(0, 64, z1, 0)

        def z3(t, _):
            pltpu.sync_copy(zbuf, acc_sh.at[pl.ds(s * NPT + t * 64, 64)])
            return 0
        lax.fori_loop(0, NPT // 64, z3, 0)

        @pl.when(s == 0)
        def _():
            pltpu.sync_copy(zbuf.at[pl.ds(0, 8)], acc_sh.at[pl.ds(NSC, 8)])

        def batch(b, _):
            def bld(g, _):
                pk = list_v[pl.ds(b * EB + g * 16, 16)]
                valid = (b * EB + g * 16 + lanes) < cnt
                sv = pk & jnp.int32(16383)
                t = lax.shift_right_logical(pk, 14)
                idxb[pl.ds(g * 16, 16)] = jnp.where(valid, sv, 0)
                sclsb[pl.ds(g * 16, 16)] = jnp.where(valid, s * NPT + t, NSC)
                return 0
            lax.fori_loop(0, EB // 16, bld, 0)
            pltpu.async_copy(a_hbm.at[idxb], rows, sem).wait()
            pltpu.sync_copy(rows, acc_sh.at[sclsb], add=True)
            return 0
        lax.fori_loop(0, (cnt + EB - 1) // EB, batch, 0)

        def outp(t, _):
            pltpu.sync_copy(acc_sh.at[pl.ds(s * NPT + t * 64, 64)],
                            rows.at[pl.ds(0, 64)])
            pltpu.sync_copy(rows.at[pl.ds(0, 64)],
                            agg_hbm.at[pl.ds(base + t * 64, 64)])
            return 0
        lax.fori_loop(0, NPT // 64, outp, 0)

    return k(lists, cnts, a)


ECH_A = 4000
EB_A = 32


def _sc_attention_top(src, dst, x, weff, beff16, sj_pad):
    mesh = plsc.VectorSubcoreMesh(core_axis_name="c", subcore_axis_name="s")
    ninf = jnp.float32(float("-inf"))

    @functools.partial(
        pl.kernel,
        out_type=[
            jax.ShapeDtypeStruct((NPAD, 256), jnp.float32),
            jax.ShapeDtypeStruct((NPAD,), jnp.float32),
            jax.ShapeDtypeStruct((NW, CAP), jnp.int32),
            jax.ShapeDtypeStruct((NW, 16), jnp.int32),
        ],
        mesh=mesh,
        compiler_params=pltpu.CompilerParams(needs_layout_passes=False),
        scratch_types=[
            pltpu.VMEM((CAP,), jnp.int32),
            pltpu.VMEM((ECH_A,), jnp.int32),
            pltpu.VMEM((ECH_A,), jnp.int32),
            pltpu.VMEM((NPAD,), jnp.float32),
            pltpu.VMEM((NPT, 256), jnp.float32),
            pltpu.VMEM((EB_A, 256), jnp.float32),
            pltpu.VMEM((EB_A, 256), jnp.float32),
            pltpu.VMEM((NPT,), jnp.float32),
            pltpu.VMEM((NPT,), jnp.float32),
            pltpu.VMEM((NPT,), jnp.float32),
            pltpu.VMEM((EB_A,), jnp.float32),
            pltpu.VMEM((EB_A,), jnp.float32),
            pltpu.VMEM((EB_A,), jnp.int32),
            pltpu.VMEM((EB_A,), jnp.int32),
            pltpu.VMEM((EB_A,), jnp.int32),
            pltpu.VMEM((EB_A,), jnp.int32),
            pltpu.VMEM((256,), jnp.float32),
            pltpu.VMEM((16,), jnp.float32),
            pltpu.VMEM((16,), jnp.int32),
            pltpu.SemaphoreType.DMA,
            pltpu.SemaphoreType.DMA,
        ],
    )
    def k(src_hbm, dst_hbm, x_hbm, weff_hbm, beff_hbm, sj_hbm,
          xtop_hbm, deg_hbm, lists_hbm, cnts_hbm,
          list_v, srcc, dstc, sj_v, acc, rows0, rows1, sq_v, ssum_v, deg_v,
          alphab0, alphab1, idxb0, idxb1, tb0, tb1,
          weff_v, beff_v, cntb, sem0, sem1):
        c = lax.axis_index("c")
        s = lax.axis_index("s")
        wid = c * 16 + s
        base = wid * NPT
        lanes = lax.iota(jnp.int32, 16)
        zeros16 = jnp.zeros((16,), jnp.float32)
        ninf16 = jnp.full((16,), ninf, jnp.float32)

        pltpu.sync_copy(sj_hbm, sj_v)
        pltpu.sync_copy(weff_hbm, weff_v)
        pltpu.sync_copy(beff_hbm, beff_v)

        def z1(g, _):
            deg_v[pl.ds(g * 16, 16)] = zeros16
            ssum_v[pl.ds(g * 16, 16)] = zeros16
            sq_v[pl.ds(g * 16, 16)] = zeros16
            return 0
        lax.fori_loop(0, NPT // 16, z1, 0)

        def z2(r, _):
            for kk in range(16):
                acc[r, pl.ds(16 * kk, 16)] = ninf16
            return 0
        lax.fori_loop(0, NPT, z2, 0)

        def chunk(ci, cnt):
            pltpu.sync_copy(src_hbm.at[pl.ds(ci * ECH_A, ECH_A)], srcc)
            pltpu.sync_copy(dst_hbm.at[pl.ds(ci * ECH_A, ECH_A)], dstc)

            def inner(i, cnt):
                sv = srcc[pl.ds(i * 16, 16)]
                dv = dstc[pl.ds(i * 16, 16)]
                t = dv - base
                m = (t >= 0) & (t < NPT)
                packed = sv | lax.shift_left(t, 14)
                cnt_c = jnp.minimum(cnt, CAP - 16)
                cs = plsc.cumsum(m.astype(jnp.int32))
                plsc.store_scatter(list_v, [cnt_c + cs - 1], packed, mask=m)
                return cnt_c + jnp.sum(m.astype(jnp.int32))
            return lax.fori_loop(0, ECH_A // 16, inner, cnt)
        cnt = lax.fori_loop(0, E_TOP // ECH_A, chunk, jnp.int32(0))

        pltpu.sync_copy(list_v, lists_hbm.at[wid])
        cntb[pl.ds(0, 16)] = jnp.full((16,), cnt, jnp.int32)
        pltpu.sync_copy(cntb, cnts_hbm.at[wid])

        ones16 = jnp.ones((16,), jnp.float32)

        def dacc(g, _):
            pk = list_v[pl.ds(g * 16, 16)]
            valid = (g * 16 + lanes) < cnt
            t = lax.shift_right_logical(pk, 14)
            t = jnp.where(valid, t, NPT - 1)
            plsc.addupdate_scatter(deg_v, [t], ones16, mask=valid)
            return 0
        lax.fori_loop(0, (cnt + 15) // 16, dacc, 0)

        nb = (cnt + EB_A - 1) // EB_A

        def bldA(b, idxb, tb):
            def g_(g, _):
                pk = list_v[pl.ds(b * EB_A + g * 16, 16)]
                valid = (b * EB_A + g * 16 + lanes) < cnt
                idxb[pl.ds(g * 16, 16)] = jnp.where(valid, pk & jnp.int32(16383), 0)
                tb[pl.ds(g * 16, 16)] = lax.shift_right_logical(pk, 14)
                return 0
            lax.fori_loop(0, EB_A // 16, g_, 0)

        def procA(b, rows, tb):
            nr = jnp.minimum(EB_A, cnt - b * EB_A)

            def rowacc(r, _):
                ts = plsc.load_gather(tb, [jnp.full((16,), r, jnp.int32)])[0]
                for kk in range(16):
                    sl = pl.ds(16 * kk, 16)
                    acc[ts, sl] = jnp.maximum(acc[ts, sl], rows[r, sl])
                return 0
            lax.fori_loop(0, nr, rowacc, 0)

        @pl.when(nb > 0)
        def _():
            bldA(0, idxb0, tb0)
            pltpu.async_copy(x_hbm.at[idxb0], rows0, sem0)

        def pairA(p, _):
            b0 = 2 * p
            b1 = b0 + 1

            @pl.when(b1 < nb)
            def _():
                bldA(b1, idxb1, tb1)
                pltpu.async_copy(x_hbm.at[idxb1], rows1, sem1)
            pltpu.make_async_copy(x_hbm.at[idxb0], rows0, sem0).wait()
            procA(b0, rows0, tb0)

            @pl.when(b0 + 2 < nb)
            def _():
                bldA(b0 + 2, idxb0, tb0)
                pltpu.async_copy(x_hbm.at[idxb0], rows0, sem0)

            @pl.when(b1 < nb)
            def _():
                pltpu.make_async_copy(x_hbm.at[idxb1], rows1, sem1).wait()
                procA(b1, rows1, tb1)
            return 0
        lax.fori_loop(0, (nb + 1) // 2, pairA, 0)

        beff_s = beff_v[pl.ds(0, 16)][0]

        def nodeB(r, _):
            accum = zeros16
            for kk in range(16):
                sl = pl.ds(16 * kk, 16)
                row = acc[r, sl]
                rowf = jnp.where(row > ninf, row, 0.0)
                accum = accum + rowf * weff_v[sl]
            sq_s = jnp.sum(accum) + beff_s
            plsc.store_scatter(sq_v, [jnp.full((16,), r, jnp.int32)],
                               jnp.full((16,), sq_s, jnp.float32),
                               mask=lanes < 1)
            return 0
        lax.fori_loop(0, NPT, nodeB, 0)

        def grpB2(g, _):
            pk = list_v[pl.ds(g * 16, 16)]
            valid = (g * 16 + lanes) < cnt
            sv = jnp.where(valid, pk & jnp.int32(16383), 0)
            t = jnp.where(valid, lax.shift_right_logical(pk, 14), 0)
            sc = plsc.load_gather(sj_v, [sv]) + plsc.load_gather(sq_v, [t])
            sc = jnp.where(sc >= 0, sc, NEG_SLOPE * sc)
            ev = jnp.exp(sc)
            plsc.addupdate_scatter(ssum_v, [t], ev, mask=valid)
            return 0
        lax.fori_loop(0, (cnt + 15) // 16, grpB2, 0)

        def zc(r, _):
            for kk in range(16):
                acc[r, pl.ds(16 * kk, 16)] = zeros16
            return 0
        lax.fori_loop(0, NPT, zc, 0)

        def bldC(b, idxb, tb, alphab):
            def g_(g, _):
                pk = list_v[pl.ds(b * EB_A + g * 16, 16)]
                valid = (b * EB_A + g * 16 + lanes) < cnt
                sv = jnp.where(valid, pk & jnp.int32(16383), 0)
                t = jnp.where(valid, lax.shift_right_logical(pk, 14), 0)
                sc = plsc.load_gather(sj_v, [sv]) + plsc.load_gather(sq_v, [t])
                sc = jnp.where(sc >= 0, sc, NEG_SLOPE * sc)
                ev = jnp.exp(sc)
                ssv = plsc.load_gather(ssum_v, [t])
                alphab[pl.ds(g * 16, 16)] = ev / (ssv + 1e-16)
                idxb[pl.ds(g * 16, 16)] = sv
                tb[pl.ds(g * 16, 16)] = t
                return 0
            lax.fori_loop(0, EB_A // 16, g_, 0)

        def procC(b, rows, tb, alphab):
            nr = jnp.minimum(EB_A, cnt - b * EB_A)

            def rowadd(r, _):
                av = plsc.load_gather(alphab, [jnp.full((16,), r, jnp.int32)])
                ts = plsc.load_gather(tb, [jnp.full((16,), r, jnp.int32)])[0]
                for kk in range(16):
                    sl = pl.ds(16 * kk, 16)
                    acc[ts, sl] = acc[ts, sl] + rows[r, sl] * av
                return 0
            lax.fori_loop(0, nr, rowadd, 0)

        @pl.when(nb > 0)
        def _():
            bldC(0, idxb0, tb0, alphab0)
            pltpu.async_copy(x_hbm.at[idxb0], rows0, sem0)

        def pairC(p, _):
            b0 = 2 * p
            b1 = b0 + 1

            @pl.when(b1 < nb)
            def _():
                bldC(b1, idxb1, tb1, alphab1)
                pltpu.async_copy(x_hbm.at[idxb1], rows1, sem1)
            pltpu.make_async_copy(x_hbm.at[idxb0], rows0, sem0).wait()
            procC(b0, rows0, tb0, alphab0)

            @pl.when(b0 + 2 < nb)
            def _():
                bldC(b0 + 2, idxb0, tb0, alphab0)
                pltpu.async_copy(x_hbm.at[idxb0], rows0, sem0)

            @pl.when(b1 < nb)
            def _():
                pltpu.make_async_copy(x_hbm.at[idxb1], rows1, sem1).wait()
                procC(b1, rows1, tb1, alphab1)
            return 0
        lax.fori_loop(0, (nb + 1) // 2, pairC, 0)

        def outX(t, _):
            pltpu.sync_copy(acc.at[pl.ds(t * EB_A, EB_A)],
                            xtop_hbm.at[pl.ds(base + t * EB_A, EB_A)])
            return 0
        lax.fori_loop(0, NPT // EB_A, outX, 0)
        pltpu.sync_copy(deg_v, deg_hbm.at[pl.ds(base, NPT)])

    return k(src, dst, x, weff, beff16, sj_pad)


TB = 1024
NB = NPAD // TB
NINF = float("-inf")


def _tc_seq_body(xm_ref, x_ref, xp_ref, weffs_ref, wjs_ref, wjt_ref, c_ref,
                 xseq_ref, sjt_ref):
    i = pl.program_id(0)
    xb = x_ref[...]
    xm1 = jnp.concatenate([xm_ref[TB - 1:TB, :], xb[:TB - 1, :]], axis=0)
    xp1 = jnp.concatenate([xb[1:, :], xp_ref[0:1, :]], axis=0)
    gid = i * TB + jax.lax.broadcasted_iota(jnp.int32, (TB, 1), 0)
    v1 = gid >= 1
    v2 = gid <= N - 2
    m_seq = jnp.maximum(jnp.where(v1, xm1, NINF), jnp.where(v2, xp1, NINF))
    beff = c_ref[0, 0]
    sq = jnp.dot(m_seq, weffs_ref[...], preferred_element_type=jnp.float32) + beff
    sjm1 = jnp.dot(xm1, wjs_ref[...], preferred_element_type=jnp.float32)
    sjp1 = jnp.dot(xp1, wjs_ref[...], preferred_element_type=jnp.float32)
    t1 = sq + sjm1
    t1 = jnp.where(t1 >= 0, t1, NEG_SLOPE * t1)
    t2 = sq + sjp1
    t2 = jnp.where(t2 >= 0, t2, NEG_SLOPE * t2)
    mx = jnp.maximum(jnp.where(v1, t1, NINF), jnp.where(v2, t2, NINF))
    e1 = jnp.where(v1, jnp.exp(t1 - mx), 0.0)
    e2 = jnp.where(v2, jnp.exp(t2 - mx), 0.0)
    ssum = e1 + e2 + 1e-16
    xseq_ref[...] = xm1 * (e1 / ssum) + xp1 * (e2 / ssum)
    sjt_ref[...] = jnp.dot(xb, wjt_ref[...], preferred_element_type=jnp.float32).reshape(1, TB)


def _tc_seq(xpad, weffs, wjs, wjt, beff_seq):
    carr = jnp.full((1, 128), beff_seq, jnp.float32)
    return pl.pallas_call(
        _tc_seq_body,
        grid=(NB,),
        in_specs=[
            pl.BlockSpec((TB, 256), lambda i: (jnp.maximum(i - 1, 0), 0)),
            pl.BlockSpec((TB, 256), lambda i: (i, 0)),
            pl.BlockSpec((TB, 256), lambda i: (jnp.minimum(i + 1, NB - 1), 0)),
            pl.BlockSpec((256, 1), lambda i: (0, 0)),
            pl.BlockSpec((256, 1), lambda i: (0, 0)),
            pl.BlockSpec((256, 1), lambda i: (0, 0)),
            pl.BlockSpec((1, 128), lambda i: (0, 0)),
        ],
        out_specs=[
            pl.BlockSpec((TB, 256), lambda i: (i, 0)),
            pl.BlockSpec((1, TB), lambda i: (0, i)),
        ],
        out_shape=[
            jax.ShapeDtypeStruct((NPAD, 256), jnp.float32),
            jax.ShapeDtypeStruct((1, NPAD), jnp.float32),
        ],
    )(xpad, xpad, xpad, weffs, wjs, wjt, carr)


def _tc_abc_body(xt_ref, xs_ref, wt_ref, ws_ref, b_ref, o_ref):
    o_ref[...] = (jnp.dot(xt_ref[...], wt_ref[...], preferred_element_type=jnp.float32)
                  + jnp.dot(xs_ref[...], ws_ref[...], preferred_element_type=jnp.float32)
                  + b_ref[...])


def _tc_abc(xtop, xseq, w_top_half, w_seq_half, b_all):
    return pl.pallas_call(
        _tc_abc_body,
        grid=(NB,),
        in_specs=[
            pl.BlockSpec((TB, 256), lambda i: (i, 0)),
            pl.BlockSpec((TB, 256), lambda i: (i, 0)),
            pl.BlockSpec((256, 768), lambda i: (0, 0)),
            pl.BlockSpec((256, 768), lambda i: (0, 0)),
            pl.BlockSpec((1, 768), lambda i: (0, 0)),
        ],
        out_specs=pl.BlockSpec((TB, 768), lambda i: (i, 0)),
        out_shape=jax.ShapeDtypeStruct((NPAD, 768), jnp.float32),
    )(xtop, xseq, w_top_half, w_seq_half, b_all)


def _tc_fit_body(am_ref, a_ref, ap_ref, agg_ref, degb_ref, fit_ref, msum_ref):
    i = pl.program_id(0)
    abc = a_ref[...]
    gid = i * TB + jax.lax.broadcasted_iota(jnp.int32, (TB, 1), 0)
    v1 = (gid >= 1).astype(jnp.float32)
    v2 = (gid <= N - 2).astype(jnp.float32)
    fit_t = agg_ref[...] - degb_ref[...] * abc[:, 128:256] + abc[:, 256:384]
    a_s = abc[:, 384:512]
    asm1 = jnp.concatenate([am_ref[TB - 1:TB, 384:512], a_s[:TB - 1, :]], axis=0)
    asp1 = jnp.concatenate([a_s[1:, :], ap_ref[0:1, 384:512]], axis=0)
    sum_s = v1 * asm1 + v2 * asp1
    fit_s = sum_s - (v1 + v2) * abc[:, 512:640] + abc[:, 640:768]
    fit = jnp.concatenate([fit_t, fit_s], axis=1)
    fit_ref[...] = fit
    valid = gid < N

    @pl.when(i == 0)
    def _():
        msum_ref[...] = jnp.zeros_like(msum_ref)
    msum_ref[...] += jnp.sum(jnp.where(valid, fit, 0.0), axis=0, keepdims=True)


def _tc_fit(abc, agg, degb):
    return pl.pallas_call(
        _tc_fit_body,
        grid=(NB,),
        in_specs=[
            pl.BlockSpec((TB, 768), lambda i: (jnp.maximum(i - 1, 0), 0)),
            pl.BlockSpec((TB, 768), lambda i: (i, 0)),
            pl.BlockSpec((TB, 768), lambda i: (jnp.minimum(i + 1, NB - 1), 0)),
            pl.BlockSpec((TB, 128), lambda i: (i, 0)),
            pl.BlockSpec((TB, 128), lambda i: (i, 0)),
        ],
        out_specs=[
            pl.BlockSpec((TB, 256), lambda i: (i, 0)),
            pl.BlockSpec((1, 256), lambda i: (0, 0)),
        ],
        out_shape=[
            jax.ShapeDtypeStruct((NPAD, 256), jnp.float32),
            jax.ShapeDtypeStruct((1, 256), jnp.float32),
        ],
    )(abc, abc, abc, agg, degb)


def _tc_out_body(fit_ref, msum_ref, f1_ref, b1_ref, f2_ref, b2_ref,
                 wf_ref, bf_ref, o_ref):
    sqz = msum_ref[...] * jnp.float32(1.0 / N)
    s1 = jnp.maximum(jnp.dot(sqz, f1_ref[...], preferred_element_type=jnp.float32)
                     + b1_ref[...], 0.0)
    z = jnp.dot(s1, f2_ref[...], preferred_element_type=jnp.float32) + b2_ref[...]
    s2 = 1.0 / (1.0 + jnp.exp(-z))
    fit = jnp.maximum(fit_ref[...] * s2, 0.0)
    o_ref[...] = jnp.dot(fit, wf_ref[...], preferred_element_type=jnp.float32) + bf_ref[...]


def _tc_out(fit, msum, f1, b1, f2, b2, wf, bf):
    return pl.pallas_call(
        _tc_out_body,
        grid=(NB,),
        in_specs=[
            pl.BlockSpec((TB, 256), lambda i: (i, 0)),
            pl.BlockSpec((1, 256), lambda i: (0, 0)),
            pl.BlockSpec((256, 64), lambda i: (0, 0)),
            pl.BlockSpec((1, 64), lambda i: (0, 0)),
            pl.BlockSpec((64, 256), lambda i: (0, 0)),
            pl.BlockSpec((1, 256), lambda i: (0, 0)),
            pl.BlockSpec((256, 256), lambda i: (0, 0)),
            pl.BlockSpec((1, 256), lambda i: (0, 0)),
        ],
        out_specs=pl.BlockSpec((TB, 256), lambda i: (i, 0)),
        out_shape=jax.ShapeDtypeStruct((NPAD, 256), jnp.float32),
    )(fit, msum, f1, b1, f2, b2, wf, bf)


def kernel(x, edge_index,
           lin_top_W, lin_top_b, att_top_W, att_top_b,
           lin_seq_W, lin_seq_b, att_seq_W, att_seq_b,
           le_top_W1, le_top_b1, le_top_W2, le_top_W3, le_top_b3,
           le_seq_W1, le_seq_b1, le_seq_W2, le_seq_W3, le_seq_b3,
           se_fc1_W, se_fc1_b, se_fc2_W, se_fc2_b,
           final_W, final_b):
    n = x.shape[0]
    xpad = jnp.concatenate([x, jnp.zeros((NPAD - n, D), jnp.float32)], axis=0)

    wq_t = att_top_W[:D, 0]
    weff_t = lin_top_W @ wq_t
    beff16 = jnp.full((16,), lin_top_b @ wq_t + att_top_b[0], jnp.float32)
    wq_s = att_seq_W[:D, 0]
    weffs = (lin_seq_W @ wq_s).reshape(D, 1)
    beff_s = lin_seq_b @ wq_s + att_seq_b[0]
    wjs = att_seq_W[D:, :]
    wjt = att_top_W[D:, :]

    xseq_pad, sjt = _tc_seq(xpad, weffs, wjs, wjt, beff_s)
    xtop_pad, deg_pad, lists, cnts = _sc_attention_top(
        edge_index[0], edge_index[1], x, weff_t, beff16, sjt.reshape(NPAD))

    w_top_half = jnp.concatenate(
        [le_top_W1[:D], le_top_W2[:D], le_top_W3[:D],
         le_seq_W1[:D], le_seq_W2[:D], le_seq_W3[:D]], axis=1)
    w_seq_half = jnp.concatenate(
        [le_top_W1[D:], le_top_W2[D:], le_top_W3[D:],
         le_seq_W1[D:], le_seq_W2[D:], le_seq_W3[D:]], axis=1)
    z128 = jnp.zeros((128,), jnp.float32)
    b_all = jnp.concatenate(
        [le_top_b1, z128, le_top_b3, le_seq_b1, z128, le_seq_b3]).reshape(1, 768)
    abc = _tc_abc(xtop_pad, xseq_pad, w_top_half, w_seq_half, b_all)

    agg_pad = _sc_leconv_agg(lists, cnts, abc[:, :128])[0]

    degb = jnp.broadcast_to(deg_pad[:, None], (NPAD, 128))
    fit_pad, msum = _tc_fit(abc, agg_pad, degb)

    out_pad = _tc_out(fit_pad, msum, se_fc1_W, se_fc1_b.reshape(1, -1),
                      se_fc2_W, se_fc2_b.reshape(1, -1),
                      final_W, final_b.reshape(1, -1))
    return out_pad[:n]


# speedup vs baseline: 4.0077x; 1.0361x over previous
import functools

import jax
import jax.numpy as jnp
from jax import lax
from jax.experimental import pallas as pl
from jax.experimental.pallas import tpu as pltpu
from jax.experimental.pallas import tpu_sc as plsc

N = 10000
D = 256
NEG_SLOPE = 0.2

NW = 32
NPT = 320
NPAD = NW * NPT
NSC = 16 * NPT
CAP = 10240
ECH = 8000
EB = 128
E_TOP = 160000


def _sc_leconv_agg(lists, cnts, a):
    F = a.shape[1]
    mesh = plsc.VectorSubcoreMesh(core_axis_name="c", subcore_axis_name="s")

    @functools.partial(
        pl.kernel,
        out_type=[
            jax.ShapeDtypeStruct((NPAD, F), jnp.float32),
        ],
        mesh=mesh,
        compiler_params=pltpu.CompilerParams(needs_layout_passes=False),
        scratch_types=[
            pltpu.VMEM((CAP,), jnp.int32),
            pltpu.VMEM((16,), jnp.int32),
            pltpu.VMEM((EB,), jnp.int32),
            pltpu.VMEM((EB,), jnp.int32),
            pltpu.VMEM((EB, 128), jnp.float32),
            pltpu.VMEM((64, 128), jnp.float32),
            pltpu.VMEM_SHARED((NSC + 8, 128), jnp.float32),
            pltpu.SemaphoreType.DMA,
        ],
    )
    def k(lists_hbm, cnts_hbm, a_hbm, agg_hbm,
          list_v, cntb, idxb, sclsb, rows, zbuf, acc_sh, sem):
        c = lax.axis_index("c")
        s = lax.axis_index("s")
        wid = c * 16 + s
        base = wid * NPT
        lanes = lax.iota(jnp.int32, 16)
        zeros16 = jnp.zeros((16,), jnp.float32)

        pltpu.sync_copy(lists_hbm.at[wid], list_v)
        pltpu.sync_copy(cnts_hbm.at[wid], cntb)
        cnt = cntb[pl.ds(0, 16)][0]

        def z1(r, _):
            for kk in range(128 // 16):
                zbuf[r, pl.ds(16 * kk, 16)] = zeros16
            return 0
        lax.fori_loop(0, 64, z1, 0)

        def z3(t, _):
            pltpu.sync_copy(zbuf, acc_sh.at[pl.ds(s * NPT + t * 64, 64)])
            return 0
        lax.fori_loop(0, NPT // 64, z3, 0)

        @pl.when(s == 0)
        def _():
            pltpu.sync_copy(zbuf.at[pl.ds(0, 8)], acc_sh.at[pl.ds(NSC, 8)])

        def batch(b, _):
            def bld(g, _):
                pk = list_v[pl.ds(b * EB + g * 16, 16)]
                valid = (b * EB + g * 16 + lanes) < cnt
                sv = pk & jnp.int32(16383)
                t = lax.shift_right_logical(pk, 14)
                idxb[pl.ds(g * 16, 16)] = jnp.where(valid, sv, 0)
                sclsb[pl.ds(g * 16, 16)] = jnp.where(valid, s * NPT + t, NSC)
                return 0
            lax.fori_loop(0, EB // 16, bld, 0)
            pltpu.async_copy(a_hbm.at[idxb], rows, sem).wait()
            pltpu.sync_copy(rows, acc_sh.at[sclsb], add=True)
            return 0
        lax.fori_loop(0, (cnt + EB - 1) // EB, batch, 0)

        def outp(t, _):
            pltpu.sync_copy(acc_sh.at[pl.ds(s * NPT + t * 64, 64)],
                            rows.at[pl.ds(0, 64)])
            pltpu.sync_copy(rows.at[pl.ds(0, 64)],
                            agg_hbm.at[pl.ds(base + t * 64, 64)])
            return 0
        lax.fori_loop(0, NPT // 64, outp, 0)

    return k(lists, cnts, a)


ECH_A = 2000
EB_A = 32


def _sc_attention_top(src, dst, x, weff, beff16, sj_pad):
    mesh = plsc.VectorSubcoreMesh(core_axis_name="c", subcore_axis_name="s")
    ninf = jnp.float32(float("-inf"))

    @functools.partial(
        pl.kernel,
        out_type=[
            jax.ShapeDtypeStruct((NPAD, 256), jnp.float32),
            jax.ShapeDtypeStruct((NPAD,), jnp.float32),
            jax.ShapeDtypeStruct((NW, CAP), jnp.int32),
            jax.ShapeDtypeStruct((NW, 16), jnp.int32),
        ],
        mesh=mesh,
        compiler_params=pltpu.CompilerParams(needs_layout_passes=False),
        scratch_types=[
            pltpu.VMEM((CAP,), jnp.int32),
            pltpu.VMEM((ECH_A,), jnp.int32),
            pltpu.VMEM((ECH_A,), jnp.int32),
            pltpu.VMEM((ECH_A,), jnp.int32),
            pltpu.VMEM((ECH_A,), jnp.int32),
            pltpu.VMEM((NPAD,), jnp.float32),
            pltpu.VMEM((NPT, 256), jnp.float32),
            pltpu.VMEM((EB_A, 256), jnp.float32),
            pltpu.VMEM((EB_A, 256), jnp.float32),
            pltpu.VMEM((NPT,), jnp.float32),
            pltpu.VMEM((NPT,), jnp.float32),
            pltpu.VMEM((NPT,), jnp.float32),
            pltpu.VMEM((EB_A,), jnp.float32),
            pltpu.VMEM((EB_A,), jnp.float32),
            pltpu.VMEM((EB_A,), jnp.int32),
            pltpu.VMEM((EB_A,), jnp.int32),
            pltpu.VMEM((EB_A,), jnp.int32),
            pltpu.VMEM((EB_A,), jnp.int32),
            pltpu.VMEM((256,), jnp.float32),
            pltpu.VMEM((16,), jnp.float32),
            pltpu.VMEM((16,), jnp.int32),
            pltpu.SemaphoreType.DMA,
            pltpu.SemaphoreType.DMA,
        ],
    )
    def k(src_hbm, dst_hbm, x_hbm, weff_hbm, beff_hbm, sj_hbm,
          xtop_hbm, deg_hbm, lists_hbm, cnts_hbm,
          list_v, srcc0, dstc0, srcc1, dstc1, sj_v, acc, rows0, rows1,
          sq_v, ssum_v, deg_v,
          alphab0, alphab1, idxb0, idxb1, tb0, tb1,
          weff_v, beff_v, cntb, sem0, sem1):
        c = lax.axis_index("c")
        s = lax.axis_index("s")
        wid = c * 16 + s
        base = wid * NPT
        lanes = lax.iota(jnp.int32, 16)
        zeros16 = jnp.zeros((16,), jnp.float32)
        ninf16 = jnp.full((16,), ninf, jnp.float32)

        pltpu.sync_copy(sj_hbm, sj_v)
        pltpu.sync_copy(weff_hbm, weff_v)
        pltpu.sync_copy(beff_hbm, beff_v)

        def z1(g, _):
            deg_v[pl.ds(g * 16, 16)] = zeros16
            ssum_v[pl.ds(g * 16, 16)] = zeros16
            sq_v[pl.ds(g * 16, 16)] = zeros16
            return 0
        lax.fori_loop(0, NPT // 16, z1, 0)

        def z2(r, _):
            for kk in range(16):
                acc[r, pl.ds(16 * kk, 16)] = ninf16
            return 0
        lax.fori_loop(0, NPT, z2, 0)

        NCH = E_TOP // ECH_A

        def scan_buf(sbuf, dbuf, cnt):
            def inner(i, cnt):
                sv = sbuf[pl.ds(i * 16, 16)]
                dv = dbuf[pl.ds(i * 16, 16)]
                t = dv - base
                m = (t >= 0) & (t < NPT)
                packed = sv | lax.shift_left(t, 14)
                cnt_c = jnp.minimum(cnt, CAP - 16)
                cs = plsc.cumsum(m.astype(jnp.int32))
                plsc.store_scatter(list_v, [cnt_c + cs - 1], packed, mask=m)
                return cnt_c + jnp.sum(m.astype(jnp.int32))
            return lax.fori_loop(0, ECH_A // 16, inner, cnt)

        pltpu.async_copy(src_hbm.at[pl.ds(0, ECH_A)], srcc0, sem0)
        pltpu.async_copy(dst_hbm.at[pl.ds(0, ECH_A)], dstc0, sem0)

        def chunkpair(p, cnt):
            c0 = 2 * p
            c1 = c0 + 1
            pltpu.async_copy(src_hbm.at[pl.ds(c1 * ECH_A, ECH_A)], srcc1, sem1)
            pltpu.async_copy(dst_hbm.at[pl.ds(c1 * ECH_A, ECH_A)], dstc1, sem1)
            pltpu.make_async_copy(src_hbm.at[pl.ds(c0 * ECH_A, ECH_A)], srcc0, sem0).wait()
            pltpu.make_async_copy(dst_hbm.at[pl.ds(c0 * ECH_A, ECH_A)], dstc0, sem0).wait()
            cnt = scan_buf(srcc0, dstc0, cnt)

            @pl.when(c0 + 2 < NCH)
            def _():
                pltpu.async_copy(src_hbm.at[pl.ds((c0 + 2) * ECH_A, ECH_A)], srcc0, sem0)
                pltpu.async_copy(dst_hbm.at[pl.ds((c0 + 2) * ECH_A, ECH_A)], dstc0, sem0)
            pltpu.make_async_copy(src_hbm.at[pl.ds(c1 * ECH_A, ECH_A)], srcc1, sem1).wait()
            pltpu.make_async_copy(dst_hbm.at[pl.ds(c1 * ECH_A, ECH_A)], dstc1, sem1).wait()
            return scan_buf(srcc1, dstc1, cnt)
        cnt = lax.fori_loop(0, NCH // 2, chunkpair, jnp.int32(0))

        pltpu.sync_copy(list_v, lists_hbm.at[wid])
        cntb[pl.ds(0, 16)] = jnp.full((16,), cnt, jnp.int32)
        pltpu.sync_copy(cntb, cnts_hbm.at[wid])

        ones16 = jnp.ones((16,), jnp.float32)

        def dacc(g, _):
            pk = list_v[pl.ds(g * 16, 16)]
            valid = (g * 16 + lanes) < cnt
            t = lax.shift_right_logical(pk, 14)
            t = jnp.where(valid, t, NPT - 1)
            plsc.addupdate_scatter(deg_v, [t], ones16, mask=valid)
            return 0
        lax.fori_loop(0, (cnt + 15) // 16, dacc, 0)

        nb = (cnt + EB_A - 1) // EB_A

        def bldA(b, idxb, tb):
            def g_(g, _):
                pk = list_v[pl.ds(b * EB_A + g * 16, 16)]
                valid = (b * EB_A + g * 16 + lanes) < cnt
                idxb[pl.ds(g * 16, 16)] = jnp.where(valid, pk & jnp.int32(16383), 0)
                tb[pl.ds(g * 16, 16)] = lax.shift_right_logical(pk, 14)
                return 0
            lax.fori_loop(0, EB_A // 16, g_, 0)

        def procA(b, rows, tb):
            nr = jnp.minimum(EB_A, cnt - b * EB_A)

            def one(r):
                ts = plsc.load_gather(tb, [jnp.full((16,), r, jnp.int32)])[0]
                for kk in range(16):
                    sl = pl.ds(16 * kk, 16)
                    acc[ts, sl] = jnp.maximum(acc[ts, sl], rows[r, sl])

            def rowacc2(rr, _):
                one(rr * 2)
                one(rr * 2 + 1)
                return 0
            lax.fori_loop(0, nr // 2, rowacc2, 0)

            @pl.when(nr % 2 == 1)
            def _():
                one(nr - 1)

        @pl.when(nb > 0)
        def _():
            bldA(0, idxb0, tb0)
            pltpu.async_copy(x_hbm.at[idxb0], rows0, sem0)

        def pairA(p, _):
            b0 = 2 * p
            b1 = b0 + 1

            @pl.when(b1 < nb)
            def _():
                bldA(b1, idxb1, tb1)
                pltpu.async_copy(x_hbm.at[idxb1], rows1, sem1)
            pltpu.make_async_copy(x_hbm.at[idxb0], rows0, sem0).wait()
            procA(b0, rows0, tb0)

            @pl.when(b0 + 2 < nb)
            def _():
                bldA(b0 + 2, idxb0, tb0)
                pltpu.async_copy(x_hbm.at[idxb0], rows0, sem0)

            @pl.when(b1 < nb)
            def _():
                pltpu.make_async_copy(x_hbm.at[idxb1], rows1, sem1).wait()
                procA(b1, rows1, tb1)
            return 0
        lax.fori_loop(0, (nb + 1) // 2, pairA, 0)

        beff_s = beff_v[pl.ds(0, 16)][0]

        def nodeB(r, _):
            accum = zeros16
            for kk in range(16):
                sl = pl.ds(16 * kk, 16)
                row = acc[r, sl]
                rowf = jnp.where(row > ninf, row, 0.0)
                accum = accum + rowf * weff_v[sl]
            sq_s = jnp.sum(accum) + beff_s
            plsc.store_scatter(sq_v, [jnp.full((16,), r, jnp.int32)],
                               jnp.full((16,), sq_s, jnp.float32),
                               mask=lanes < 1)
            return 0
        lax.fori_loop(0, NPT, nodeB, 0)

        def grpB2(g, _):
            pk = list_v[pl.ds(g * 16, 16)]
            valid = (g * 16 + lanes) < cnt
            sv = jnp.where(valid, pk & jnp.int32(16383), 0)
            t = jnp.where(valid, lax.shift_right_logical(pk, 14), 0)
            sc = plsc.load_gather(sj_v, [sv]) + plsc.load_gather(sq_v, [t])
            sc = jnp.where(sc >= 0, sc, NEG_SLOPE * sc)
            ev = jnp.exp(sc)
            plsc.addupdate_scatter(ssum_v, [t], ev, mask=valid)
            return 0
        lax.fori_loop(0, (cnt + 15) // 16, grpB2, 0)

        def zc(r, _):
            for kk in range(16):
                acc[r, pl.ds(16 * kk, 16)] = zeros16
            return 0
        lax.fori_loop(0, NPT, zc, 0)

        def bldC(b, idxb, tb, alphab):
            def g_(g, _):
                pk = list_v[pl.ds(b * EB_A + g * 16, 16)]
                valid = (b * EB_A + g * 16 + lanes) < cnt
                sv = jnp.where(valid, pk & jnp.int32(16383), 0)
                t = jnp.where(valid, lax.shift_right_logical(pk, 14), 0)
                sc = plsc.load_gather(sj_v, [sv]) + plsc.load_gather(sq_v, [t])
                sc = jnp.where(sc >= 0, sc, NEG_SLOPE * sc)
                ev = jnp.exp(sc)
                ssv = plsc.load_gather(ssum_v, [t])
                alphab[pl.ds(g * 16, 16)] = ev / (ssv + 1e-16)
                idxb[pl.ds(g * 16, 16)] = sv
                tb[pl.ds(g * 16, 16)] = t
                return 0
            lax.fori_loop(0, EB_A // 16, g_, 0)

        def procC(b, rows, tb, alphab):
            nr = jnp.minimum(EB_A, cnt - b * EB_A)

            def one(r):
                av = plsc.load_gather(alphab, [jnp.full((16,), r, jnp.int32)])
                ts = plsc.load_gather(tb, [jnp.full((16,), r, jnp.int32)])[0]
                for kk in range(16):
                    sl = pl.ds(16 * kk, 16)
                    acc[ts, sl] = acc[ts, sl] + rows[r, sl] * av

            def rowadd2(rr, _):
                one(rr * 2)
                one(rr * 2 + 1)
                return 0
            lax.fori_loop(0, nr // 2, rowadd2, 0)

            @pl.when(nr % 2 == 1)
            def _():
                one(nr - 1)

        @pl.when(nb > 0)
        def _():
            bldC(0, idxb0, tb0, alphab0)
            pltpu.async_copy(x_hbm.at[idxb0], rows0, sem0)

        def pairC(p, _):
            b0 = 2 * p
            b1 = b0 + 1

            @pl.when(b1 < nb)
            def _():
                bldC(b1, idxb1, tb1, alphab1)
                pltpu.async_copy(x_hbm.at[idxb1], rows1, sem1)
            pltpu.make_async_copy(x_hbm.at[idxb0], rows0, sem0).wait()
            procC(b0, rows0, tb0, alphab0)

            @pl.when(b0 + 2 < nb)
            def _():
                bldC(b0 + 2, idxb0, tb0, alphab0)
                pltpu.async_copy(x_hbm.at[idxb0], rows0, sem0)

            @pl.when(b1 < nb)
            def _():
                pltpu.make_async_copy(x_hbm.at[idxb1], rows1, sem1).wait()
                procC(b1, rows1, tb1, alphab1)
            return 0
        lax.fori_loop(0, (nb + 1) // 2, pairC, 0)

        def outX(t, _):
            pltpu.sync_copy(acc.at[pl.ds(t * EB_A, EB_A)],
                            xtop_hbm.at[pl.ds(base + t * EB_A, EB_A)])
            return 0
        lax.fori_loop(0, NPT // EB_A, outX, 0)
        pltpu.sync_copy(deg_v, deg_hbm.at[pl.ds(base, NPT)])

    return k(src, dst, x, weff, beff16, sj_pad)


TB = 1024
NB = NPAD // TB
NINF = float("-inf")


def _tc_seq_body(xm_ref, x_ref, xp_ref, weffs_ref, wjs_ref, wjt_ref, c_ref,
                 xseq_ref, sjt_ref):
    i = pl.program_id(0)
    xb = x_ref[...]
    xm1 = jnp.concatenate([xm_ref[TB - 1:TB, :], xb[:TB - 1, :]], axis=0)
    xp1 = jnp.concatenate([xb[1:, :], xp_ref[0:1, :]], axis=0)
    gid = i * TB + jax.lax.broadcasted_iota(jnp.int32, (TB, 1), 0)
    v1 = gid >= 1
    v2 = gid <= N - 2
    m_seq = jnp.maximum(jnp.where(v1, xm1, NINF), jnp.where(v2, xp1, NINF))
    beff = c_ref[0, 0]
    sq = jnp.dot(m_seq, weffs_ref[...], preferred_element_type=jnp.float32) + beff
    sjm1 = jnp.dot(xm1, wjs_ref[...], preferred_element_type=jnp.float32)
    sjp1 = jnp.dot(xp1, wjs_ref[...], preferred_element_type=jnp.float32)
    t1 = sq + sjm1
    t1 = jnp.where(t1 >= 0, t1, NEG_SLOPE * t1)
    t2 = sq + sjp1
    t2 = jnp.where(t2 >= 0, t2, NEG_SLOPE * t2)
    mx = jnp.maximum(jnp.where(v1, t1, NINF), jnp.where(v2, t2, NINF))
    e1 = jnp.where(v1, jnp.exp(t1 - mx), 0.0)
    e2 = jnp.where(v2, jnp.exp(t2 - mx), 0.0)
    ssum = e1 + e2 + 1e-16
    xseq_ref[...] = xm1 * (e1 / ssum) + xp1 * (e2 / ssum)
    sjt_ref[...] = jnp.dot(xb, wjt_ref[...], preferred_element_type=jnp.float32).reshape(1, TB)


def _tc_seq(xpad, weffs, wjs, wjt, beff_seq):
    carr = jnp.full((1, 128), beff_seq, jnp.float32)
    return pl.pallas_call(
        _tc_seq_body,
        grid=(NB,),
        in_specs=[
            pl.BlockSpec((TB, 256), lambda i: (jnp.maximum(i - 1, 0), 0)),
            pl.BlockSpec((TB, 256), lambda i: (i, 0)),
            pl.BlockSpec((TB, 256), lambda i: (jnp.minimum(i + 1, NB - 1), 0)),
            pl.BlockSpec((256, 1), lambda i: (0, 0)),
            pl.BlockSpec((256, 1), lambda i: (0, 0)),
            pl.BlockSpec((256, 1), lambda i: (0, 0)),
            pl.BlockSpec((1, 128), lambda i: (0, 0)),
        ],
        out_specs=[
            pl.BlockSpec((TB, 256), lambda i: (i, 0)),
            pl.BlockSpec((1, TB), lambda i: (0, i)),
        ],
        out_shape=[
            jax.ShapeDtypeStruct((NPAD, 256), jnp.float32),
            jax.ShapeDtypeStruct((1, NPAD), jnp.float32),
        ],
    )(xpad, xpad, xpad, weffs, wjs, wjt, carr)


def _tc_abc_body(xt_ref, xs_ref, wt_ref, ws_ref, b_ref, o_ref):
    o_ref[...] = (jnp.dot(xt_ref[...], wt_ref[...], preferred_element_type=jnp.float32)
                  + jnp.dot(xs_ref[...], ws_ref[...], preferred_element_type=jnp.float32)
                  + b_ref[...])


def _tc_abc(xtop, xseq, w_top_half, w_seq_half, b_all):
    return pl.pallas_call(
        _tc_abc_body,
        grid=(NB,),
        in_specs=[
            pl.BlockSpec((TB, 256), lambda i: (i, 0)),
            pl.BlockSpec((TB, 256), lambda i: (i, 0)),
            pl.BlockSpec((256, 768), lambda i: (0, 0)),
            pl.BlockSpec((256, 768), lambda i: (0, 0)),
            pl.BlockSpec((1, 768), lambda i: (0, 0)),
        ],
        out_specs=pl.BlockSpec((TB, 768), lambda i: (i, 0)),
        out_shape=jax.ShapeDtypeStruct((NPAD, 768), jnp.float32),
    )(xtop, xseq, w_top_half, w_seq_half, b_all)


def _tc_fit_body(am_ref, a_ref, ap_ref, agg_ref, degb_ref, fit_ref, msum_ref):
    i = pl.program_id(0)
    abc = a_ref[...]
    gid = i * TB + jax.lax.broadcasted_iota(jnp.int32, (TB, 1), 0)
    v1 = (gid >= 1).astype(jnp.float32)
    v2 = (gid <= N - 2).astype(jnp.float32)
    fit_t = agg_ref[...] - degb_ref[...] * abc[:, 128:256] + abc[:, 256:384]
    a_s = abc[:, 384:512]
    asm1 = jnp.concatenate([am_ref[TB - 1:TB, 384:512], a_s[:TB - 1, :]], axis=0)
    asp1 = jnp.concatenate([a_s[1:, :], ap_ref[0:1, 384:512]], axis=0)
    sum_s = v1 * asm1 + v2 * asp1
    fit_s = sum_s - (v1 + v2) * abc[:, 512:640] + abc[:, 640:768]
    fit = jnp.concatenate([fit_t, fit_s], axis=1)
    fit_ref[...] = fit
    valid = gid < N

    @pl.when(i == 0)
    def _():
        msum_ref[...] = jnp.zeros_like(msum_ref)
    msum_ref[...] += jnp.sum(jnp.where(valid, fit, 0.0), axis=0, keepdims=True)


def _tc_fit(abc, agg, degb):
    return pl.pallas_call(
        _tc_fit_body,
        grid=(NB,),
        in_specs=[
            pl.BlockSpec((TB, 768), lambda i: (jnp.maximum(i - 1, 0), 0)),
            pl.BlockSpec((TB, 768), lambda i: (i, 0)),
            pl.BlockSpec((TB, 768), lambda i: (jnp.minimum(i + 1, NB - 1), 0)),
            pl.BlockSpec((TB, 128), lambda i: (i, 0)),
            pl.BlockSpec((TB, 128), lambda i: (i, 0)),
        ],
        out_specs=[
            pl.BlockSpec((TB, 256), lambda i: (i, 0)),
            pl.BlockSpec((1, 256), lambda i: (0, 0)),
        ],
        out_shape=[
            jax.ShapeDtypeStruct((NPAD, 256), jnp.float32),
            jax.ShapeDtypeStruct((1, 256), jnp.float32),
        ],
    )(abc, abc, abc, agg, degb)


def _tc_out_body(fit_ref, msum_ref, f1_ref, b1_ref, f2_ref, b2_ref,
                 wf_ref, bf_ref, o_ref):
    sqz = msum_ref[...] * jnp.float32(1.0 / N)
    s1 = jnp.maximum(jnp.dot(sqz, f1_ref[...], preferred_element_type=jnp.float32)
                     + b1_ref[...], 0.0)
    z = jnp.dot(s1, f2_ref[...], preferred_element_type=jnp.float32) + b2_ref[...]
    s2 = 1.0 / (1.0 + jnp.exp(-z))
    fit = jnp.maximum(fit_ref[...] * s2, 0.0)
    o_ref[...] = jnp.dot(fit, wf_ref[...], preferred_element_type=jnp.float32) + bf_ref[...]


def _tc_out(fit, msum, f1, b1, f2, b2, wf, bf):
    return pl.pallas_call(
        _tc_out_body,
        grid=(NB,),
        in_specs=[
            pl.BlockSpec((TB, 256), lambda i: (i, 0)),
            pl.BlockSpec((1, 256), lambda i: (0, 0)),
            pl.BlockSpec((256, 64), lambda i: (0, 0)),
            pl.BlockSpec((1, 64), lambda i: (0, 0)),
            pl.BlockSpec((64, 256), lambda i: (0, 0)),
            pl.BlockSpec((1, 256), lambda i: (0, 0)),
            pl.BlockSpec((256, 256), lambda i: (0, 0)),
            pl.BlockSpec((1, 256), lambda i: (0, 0)),
        ],
        out_specs=pl.BlockSpec((TB, 256), lambda i: (i, 0)),
        out_shape=jax.ShapeDtypeStruct((NPAD, 256), jnp.float32),
    )(fit, msum, f1, b1, f2, b2, wf, bf)


def kernel(x, edge_index,
           lin_top_W, lin_top_b, att_top_W, att_top_b,
           lin_seq_W, lin_seq_b, att_seq_W, att_seq_b,
           le_top_W1, le_top_b1, le_top_W2, le_top_W3, le_top_b3,
           le_seq_W1, le_seq_b1, le_seq_W2, le_seq_W3, le_seq_b3,
           se_fc1_W, se_fc1_b, se_fc2_W, se_fc2_b,
           final_W, final_b):
    n = x.shape[0]
    xpad = jnp.concatenate([x, jnp.zeros((NPAD - n, D), jnp.float32)], axis=0)

    wq_t = att_top_W[:D, 0]
    weff_t = lin_top_W @ wq_t
    beff16 = jnp.full((16,), lin_top_b @ wq_t + att_top_b[0], jnp.float32)
    wq_s = att_seq_W[:D, 0]
    weffs = (lin_seq_W @ wq_s).reshape(D, 1)
    beff_s = lin_seq_b @ wq_s + att_seq_b[0]
    wjs = att_seq_W[D:, :]
    wjt = att_top_W[D:, :]

    xseq_pad, sjt = _tc_seq(xpad, weffs, wjs, wjt, beff_s)
    xtop_pad, deg_pad, lists, cnts = _sc_attention_top(
        edge_index[0], edge_index[1], x, weff_t, beff16, sjt.reshape(NPAD))

    w_top_half = jnp.concatenate(
        [le_top_W1[:D], le_top_W2[:D], le_top_W3[:D],
         le_seq_W1[:D], le_seq_W2[:D], le_seq_W3[:D]], axis=1)
    w_seq_half = jnp.concatenate(
        [le_top_W1[D:], le_top_W2[D:], le_top_W3[D:],
         le_seq_W1[D:], le_seq_W2[D:], le_seq_W3[D:]], axis=1)
    z128 = jnp.zeros((128,), jnp.float32)
    b_all = jnp.concatenate(
        [le_top_b1, z128, le_top_b3, le_seq_b1, z128, le_seq_b3]).reshape(1, 768)
    abc = _tc_abc(xtop_pad, xseq_pad, w_top_half, w_seq_half, b_all)

    agg_pad = _sc_leconv_agg(lists, cnts, abc[:, :128])[0]

    degb = jnp.broadcast_to(deg_pad[:, None], (NPAD, 128))
    fit_pad, msum = _tc_fit(abc, agg_pad, degb)

    out_pad = _tc_out(fit_pad, msum, se_fc1_W, se_fc1_b.reshape(1, -1),
                      se_fc2_W, se_fc2_b.reshape(1, -1),
                      final_W, final_b.reshape(1, -1))
    return out_pad[:n]
